# baseline (speedup 1.0000x reference)
import jax
import jax.numpy as jnp
from jax.experimental import pallas as pl
from jax.experimental.pallas import tpu as pltpu

_N_EVENTS = 4096
_N_NEURONS = 1024
_TS = 256


def _normalize(ts):
    return ts / jnp.sqrt(jnp.sum(ts * ts))


def _body(all_ts_ref, w_in_ref, ch_in_ref, out_ref, w_ref, wt_ref, buf_ref,
          ch_ref, wn2_ref, rinv_ref):
    w_ref[...] = w_in_ref[...]
    wt_ref[...] = w_in_ref[...].T
    wn2_ref[...] = jnp.sum(w_in_ref[...] * w_in_ref[...], axis=1)
    rinv_ref[...] = jax.lax.rsqrt(wn2_ref[...])
    ch_ref[...] = ch_in_ref[...]
    chsum0 = jnp.sum(ch_in_ref[...])
    iota_n = jax.lax.iota(jnp.int32, _N_NEURONS)
    lane_iota = jax.lax.broadcasted_iota(jnp.int32, (_TS, 128), 1)

    tsd_0 = _normalize(all_ts_ref[pl.ds(0, 1), :])
    buf_ref[pl.ds(0, _TS), :] = tsd_0.reshape(_TS, 1)

    def _pick(sel, a, b):
        return jnp.where(sel, a, b)

    def step(t, carry):
        newrow_p, n_p, tsd = carry
        ch = ch_ref[...]
        wn2 = wn2_ref[...]
        rinv = rinv_ref[...]
        tsd_t = buf_ref[pl.ds((t % 2) * _TS, _TS), :]
        mv_fast = jnp.sum(wt_ref[...] * tsd_t, axis=0)
        mv_fast = _pick(iota_n == n_p, jnp.sum(newrow_p * tsd), mv_fast)
        chsum = chsum0 + t.astype(jnp.float32)
        gain = jnp.exp((1.0 - (ch * 1024.0) / chsum) * 0.25)
        bh_fast = gain * (mv_fast * rinv)
        n1 = jnp.argmax(bh_fast).astype(jnp.int32)
        m1 = iota_n == n1
        n2 = jnp.argmax(jnp.where(m1, -jnp.inf, bh_fast)).astype(jnp.int32)
        m2 = iota_n == n2
        row1 = _pick(n1 == n_p, newrow_p, w_ref[pl.ds(n1, 1), :])
        row2 = _pick(n2 == n_p, newrow_p, w_ref[pl.ds(n2, 1), :])
        b1 = jnp.sum(row1 * tsd) * jnp.sum(jnp.where(m1, rinv, 0.0))
        b2 = jnp.sum(row2 * tsd) * jnp.sum(jnp.where(m2, rinv, 0.0))
        bh1 = jnp.sum(jnp.where(m1, gain, 0.0)) * b1
        bh2 = jnp.sum(jnp.where(m2, gain, 0.0)) * b2
        lo1 = n1 < n2
        n_lo = _pick(lo1, n1, n2)
        n_hi = _pick(lo1, n2, n1)
        win_lo = _pick(lo1, bh1, bh2) >= _pick(lo1, bh2, bh1)
        n = _pick(win_lo, n_lo, n_hi)
        beta_n = _pick(win_lo, _pick(lo1, b1, b2), _pick(lo1, b2, b1))
        onehot = iota_n == n
        ch_n = jnp.sum(jnp.where(onehot, ch, 0.0))
        alpha = jnp.float32(0.01) / (1.0 + ch_n * jnp.float32(5e-5))
        a = alpha * beta_n
        ck = _pick(n == n1, row1, row2)
        newrow = ck + a * (tsd - ck)
        wn2_new = jnp.sum(newrow * newrow)
        wn2_ref[...] = _pick(onehot, wn2_new, wn2)
        rinv_ref[...] = _pick(onehot, jnp.max(jax.lax.rsqrt(
            jnp.full((8, 128), wn2_new, jnp.float32))), rinv)
        ch_ref[...] = _pick(onehot, ch + 1.0, ch)
        w_ref[pl.ds(n_p, 1), :] = newrow_p
        tile = (n_p // 128) * 128
        wt_ref[:, pl.ds(tile, 128)] = jnp.where(
            lane_iota == (n_p % 128), newrow_p.reshape(_TS, 1),
            wt_ref[:, pl.ds(tile, 128)])
        t_nx = jnp.minimum(t + 1, _N_EVENTS - 1)
        tsd_nx = _normalize(all_ts_ref[pl.ds(t_nx, 1), :])
        buf_ref[pl.ds(((t + 1) % 2) * _TS, _TS), :] = tsd_nx.reshape(_TS, 1)
        out_ref[0] = n
        return (newrow, n, tsd_nx)

    carry = (w_in_ref[pl.ds(0, 1), :], jnp.int32(0), tsd_0)
    carry = jax.lax.fori_loop(0, _N_EVENTS, step, carry)
    newrow_l, n_l, _ = carry
    w_ref[pl.ds(n_l, 1), :] = newrow_l


def kernel(all_ts, W, cumhisto):
    out = pl.pallas_call(
        _body,
        out_shape=jax.ShapeDtypeStruct((1,), jnp.int32),
        in_specs=[
            pl.BlockSpec(memory_space=pltpu.VMEM),
            pl.BlockSpec(memory_space=pltpu.VMEM),
            pl.BlockSpec(memory_space=pltpu.VMEM),
        ],
        out_specs=pl.BlockSpec(memory_space=pltpu.SMEM),
        scratch_shapes=[
            pltpu.VMEM((_N_NEURONS, _TS), jnp.float32),
            pltpu.VMEM((_TS, _N_NEURONS), jnp.float32),
            pltpu.VMEM((2 * _TS, 1), jnp.float32),
            pltpu.VMEM((_N_NEURONS,), jnp.float32),
            pltpu.VMEM((_N_NEURONS,), jnp.float32),
            pltpu.VMEM((_N_NEURONS,), jnp.float32),
        ],
    )(all_ts, W, cumhisto)
    return out[0]


# speedup vs baseline: 1.3708x; 1.3708x over previous
import jax
import jax.numpy as jnp
from jax.experimental import pallas as pl
from jax.experimental.pallas import tpu as pltpu

_N_EVENTS = 4096
_N_NEURONS = 1024
_TS = 256


def _body(all_ts_ref, w_in_ref, ch_in_ref, out_ref, w_ref, ch_ref, wn2_ref):
    w_ref[...] = w_in_ref[...]
    ch_ref[...] = ch_in_ref[...]
    wn2_ref[...] = jnp.sum(w_in_ref[...] * w_in_ref[...], axis=1)
    chsum0 = jnp.sum(ch_in_ref[...])
    iota_n = jax.lax.iota(jnp.int32, _N_NEURONS)

    def step(t, carry):
        ts = all_ts_ref[pl.ds(t, 1), :]
        s = jnp.sqrt(jnp.sum(ts * ts))
        tsd = ts / s
        w = w_ref[...]
        mv = jax.lax.dot_general(
            tsd, w, (((1,), (1,)), ((), ())),
            precision=jax.lax.Precision.DEFAULT,
            preferred_element_type=jnp.float32).reshape(_N_NEURONS)
        beta = mv * jax.lax.rsqrt(wn2_ref[...])
        ch = ch_ref[...]
        chsum = chsum0 + t.astype(jnp.float32)
        gain = jnp.exp((1.0 - (ch * 1024.0) / chsum) * 0.25)
        bh = gain * beta
        n = jnp.argmax(bh).astype(jnp.int32)
        onehot = iota_n == n
        ch_n = jnp.sum(jnp.where(onehot, ch, 0.0))
        beta_n = jnp.sum(jnp.where(onehot, beta, 0.0))
        alpha = jnp.float32(0.01) / (1.0 + ch_n * jnp.float32(5e-5))
        a = alpha * beta_n
        ck = w_ref[pl.ds(n, 1), :]
        newrow = ck + a * (tsd - ck)
        w_ref[pl.ds(n, 1), :] = newrow
        wn2_ref[...] = jnp.where(
            onehot, jnp.sum(newrow * newrow), wn2_ref[...])
        ch_ref[...] = jnp.where(onehot, ch + 1.0, ch)
        out_ref[0] = n
        return carry

    jax.lax.fori_loop(0, _N_EVENTS, step, jnp.int32(0))


def kernel(all_ts, W, cumhisto):
    out = pl.pallas_call(
        _body,
        out_shape=jax.ShapeDtypeStruct((1,), jnp.int32),
        in_specs=[
            pl.BlockSpec(memory_space=pltpu.VMEM),
            pl.BlockSpec(memory_space=pltpu.VMEM),
            pl.BlockSpec(memory_space=pltpu.VMEM),
        ],
        out_specs=pl.BlockSpec(memory_space=pltpu.SMEM),
        scratch_shapes=[
            pltpu.VMEM((_N_NEURONS, _TS), jnp.float32),
            pltpu.VMEM((_N_NEURONS,), jnp.float32),
            pltpu.VMEM((_N_NEURONS,), jnp.float32),
        ],
    )(all_ts, W, cumhisto)
    return out[0]


# speedup vs baseline: 1.3879x; 1.0125x over previous
import jax
import jax.numpy as jnp
from jax.experimental import pallas as pl
from jax.experimental.pallas import tpu as pltpu

_N_EVENTS = 4096
_N_NEURONS = 1024
_TS = 256


def _body(all_ts_ref, w_in_ref, ch_in_ref, out_ref, w_ref, ch_ref,
          wn2_ref, rinv_ref):
    w_ref[...] = w_in_ref[...]
    ch_ref[...] = ch_in_ref[...]
    wn2_ref[...] = jnp.sum(w_in_ref[...] * w_in_ref[...], axis=1)
    rinv_ref[...] = jax.lax.rsqrt(wn2_ref[...])
    chsum0 = jnp.sum(ch_in_ref[...])
    iota_n = jax.lax.iota(jnp.int32, _N_NEURONS)

    def step(t, carry):
        ts = all_ts_ref[pl.ds(t, 1), :]
        s = jnp.sqrt(jnp.sum(ts * ts))
        tsd = ts / s
        w = w_ref[...]
        mv = jax.lax.dot_general(
            tsd, w, (((1,), (1,)), ((), ())),
            precision=jax.lax.Precision.DEFAULT,
            preferred_element_type=jnp.float32).reshape(_N_NEURONS)
        beta = mv * rinv_ref[...]
        ch = ch_ref[...]
        chsum = chsum0 + t.astype(jnp.float32)
        gain = jnp.exp((1.0 - (ch * 1024.0) / chsum) * 0.25)
        bh = gain * beta
        alpha_v = jnp.float32(0.01) / (1.0 + ch * jnp.float32(5e-5))
        acand = alpha_v * beta
        n = jnp.argmax(bh).astype(jnp.int32)
        onehot = iota_n == n
        a = jnp.sum(jnp.where(onehot, acand, 0.0))
        ck = w_ref[pl.ds(n, 1), :]
        newrow = ck + a * (tsd - ck)
        w_ref[pl.ds(n, 1), :] = newrow
        wn2_new = jnp.sum(newrow * newrow)
        wn2_ref[...] = jnp.where(onehot, wn2_new, wn2_ref[...])
        rinv_ref[...] = jnp.where(onehot, jnp.max(jax.lax.rsqrt(
            jnp.full((8, 128), wn2_new, jnp.float32))), rinv_ref[...])
        ch_ref[...] = jnp.where(onehot, ch + 1.0, ch)
        out_ref[0] = n
        return carry

    jax.lax.fori_loop(0, _N_EVENTS, step, jnp.int32(0))


def kernel(all_ts, W, cumhisto):
    out = pl.pallas_call(
        _body,
        out_shape=jax.ShapeDtypeStruct((1,), jnp.int32),
        in_specs=[
            pl.BlockSpec(memory_space=pltpu.VMEM),
            pl.BlockSpec(memory_space=pltpu.VMEM),
            pl.BlockSpec(memory_space=pltpu.VMEM),
        ],
        out_specs=pl.BlockSpec(memory_space=pltpu.SMEM),
        scratch_shapes=[
            pltpu.VMEM((_N_NEURONS, _TS), jnp.float32),
            pltpu.VMEM((_N_NEURONS,), jnp.float32),
            pltpu.VMEM((_N_NEURONS,), jnp.float32),
            pltpu.VMEM((_N_NEURONS,), jnp.float32),
        ],
    )(all_ts, W, cumhisto)
    return out[0]


# speedup vs baseline: 1.4949x; 1.0771x over previous
import jax
import jax.numpy as jnp
from jax.experimental import pallas as pl
from jax.experimental.pallas import tpu as pltpu

_N_EVENTS = 4096
_N_NEURONS = 1024
_TS = 256


def _body(all_ts_ref, w_in_ref, ch_in_ref, out_ref, w_ref, ch_ref,
          wn2_ref, rinv_ref):
    w_ref[...] = w_in_ref[...]
    ch_ref[...] = ch_in_ref[...]
    wn2_ref[...] = jnp.sum(w_in_ref[...] * w_in_ref[...], axis=1)
    rinv_ref[...] = jax.lax.rsqrt(wn2_ref[...])
    chsum0 = jnp.sum(ch_in_ref[...])
    iota_n = jax.lax.iota(jnp.int32, _N_NEURONS)

    def step(t, carry):
        tsd = carry
        w = w_ref[...]
        mv = jax.lax.dot_general(
            tsd, w, (((1,), (1,)), ((), ())),
            precision=jax.lax.Precision.DEFAULT,
            preferred_element_type=jnp.float32).reshape(_N_NEURONS)
        beta = mv * rinv_ref[...]
        ch = ch_ref[...]
        chsum = chsum0 + t.astype(jnp.float32)
        gain = jnp.exp((1.0 - (ch * 1024.0) / chsum) * 0.25)
        bh = gain * beta
        alpha_v = jnp.float32(0.01) / (1.0 + ch * jnp.float32(5e-5))
        acand = alpha_v * beta
        n = jnp.argmax(bh).astype(jnp.int32)
        onehot = iota_n == n
        a = jnp.sum(jnp.where(onehot, acand, 0.0))
        ck = w_ref[pl.ds(n, 1), :]
        newrow = ck + a * (tsd - ck)
        w_ref[pl.ds(n, 1), :] = newrow
        wn2_new = jnp.sum(newrow * newrow)
        wn2_ref[...] = jnp.where(onehot, wn2_new, wn2_ref[...])
        rinv_ref[...] = jnp.where(onehot, jnp.max(jax.lax.rsqrt(
            jnp.full((8, 128), wn2_new, jnp.float32))), rinv_ref[...])
        ch_ref[...] = jnp.where(onehot, ch + 1.0, ch)
        out_ref[0] = n
        t_nx = jnp.minimum(t + 1, _N_EVENTS - 1)
        ts_nx = all_ts_ref[pl.ds(t_nx, 1), :]
        return ts_nx / jnp.sqrt(jnp.sum(ts_nx * ts_nx))

    ts_0 = all_ts_ref[pl.ds(0, 1), :]
    tsd_0 = ts_0 / jnp.sqrt(jnp.sum(ts_0 * ts_0))
    jax.lax.fori_loop(0, _N_EVENTS, step, tsd_0)


def kernel(all_ts, W, cumhisto):
    out = pl.pallas_call(
        _body,
        out_shape=jax.ShapeDtypeStruct((1,), jnp.int32),
        in_specs=[
            pl.BlockSpec(memory_space=pltpu.VMEM),
            pl.BlockSpec(memory_space=pltpu.VMEM),
            pl.BlockSpec(memory_space=pltpu.VMEM),
        ],
        out_specs=pl.BlockSpec(memory_space=pltpu.SMEM),
        scratch_shapes=[
            pltpu.VMEM((_N_NEURONS, _TS), jnp.float32),
            pltpu.VMEM((_N_NEURONS,), jnp.float32),
            pltpu.VMEM((_N_NEURONS,), jnp.float32),
            pltpu.VMEM((_N_NEURONS,), jnp.float32),
        ],
    )(all_ts, W, cumhisto)
    return out[0]
